# baseline (speedup 1.0000x reference)
import functools

import jax
import jax.numpy as jnp
from jax import lax
from jax.experimental import pallas as pl
from jax.experimental.pallas import tpu as pltpu
from jax.experimental.pallas import tpu_sc as plsc

B = 16384
NUM_CORES = 2
NUM_SUBCORES = 16
NW = NUM_CORES * NUM_SUBCORES
BPW = B // NW
CH = 128
NCH = BPW // CH

MF_DIM = 8
MLP_IN = 16


def _sc_gather(user3, item3, umf_t, imf_t, umlp_t, imlp_t):
    mesh = plsc.VectorSubcoreMesh(core_axis_name="c", subcore_axis_name="s")

    @functools.partial(
        pl.kernel,
        mesh=mesh,
        compiler_params=pltpu.CompilerParams(use_tc_tiling_on_sc=False),
        out_type=[
            jax.ShapeDtypeStruct((B, MF_DIM), jnp.float32),
            jax.ShapeDtypeStruct((B, MF_DIM), jnp.float32),
            jax.ShapeDtypeStruct((B, MLP_IN), jnp.float32),
            jax.ShapeDtypeStruct((B, MLP_IN), jnp.float32),
        ],
        scratch_types=[
            pltpu.VMEM((NCH, CH), jnp.int32),
            pltpu.VMEM((NCH, CH), jnp.int32),
            pltpu.VMEM((BPW, MF_DIM), jnp.float32),
            pltpu.VMEM((BPW, MF_DIM), jnp.float32),
            pltpu.VMEM((BPW, MLP_IN), jnp.float32),
            pltpu.VMEM((BPW, MLP_IN), jnp.float32),
            pltpu.SemaphoreType.DMA,
        ],
    )
    def gather_kernel(user_h, item_h, umf_h, imf_h, umlp_h, imlp_h,
                      umf_o, imf_o, umlp_o, imlp_o,
                      uidx, iidx, umf_v, imf_v, umlp_v, imlp_v, sem):
        wid = lax.axis_index("s") * NUM_CORES + lax.axis_index("c")
        pltpu.sync_copy(user_h.at[wid], uidx)
        pltpu.sync_copy(item_h.at[wid], iidx)
        copies = []
        for j in range(NCH):
            sl = pl.ds(j * CH, CH)
            copies.append(pltpu.async_copy(umf_h.at[uidx.at[j]], umf_v.at[sl], sem))
            copies.append(pltpu.async_copy(imf_h.at[iidx.at[j]], imf_v.at[sl], sem))
            copies.append(pltpu.async_copy(umlp_h.at[uidx.at[j]], umlp_v.at[sl], sem))
            copies.append(pltpu.async_copy(imlp_h.at[iidx.at[j]], imlp_v.at[sl], sem))
        for c in copies:
            c.wait()
        out_sl = pl.ds(wid * BPW, BPW)
        pltpu.sync_copy(umf_v, umf_o.at[out_sl])
        pltpu.sync_copy(imf_v, imf_o.at[out_sl])
        pltpu.sync_copy(umlp_v, umlp_o.at[out_sl])
        pltpu.sync_copy(imlp_v, imlp_o.at[out_sl])

    return gather_kernel(user3, item3, umf_t, imf_t, umlp_t, imlp_t)


def _tc_dense(umf, imf, umlp, imlp, W1, b1, W2, b2, Wf, bf):
    dn = (((1,), (0,)), ((), ()))

    def body(umf_r, imf_r, umlp_r, imlp_r, W1_r, b1_r, W2_r, b2_r, Wf_r, bf_r,
             out_r):
        gmf = umf_r[...] * imf_r[...]
        h = lax.dot_general(umlp_r[...], W1_r[0:MLP_IN, :], dn,
                            preferred_element_type=jnp.float32)
        h = h + lax.dot_general(imlp_r[...], W1_r[MLP_IN:2 * MLP_IN, :], dn,
                                preferred_element_type=jnp.float32)
        h = jnp.maximum(h + b1_r[...][None, :], 0.0)
        h = lax.dot_general(h, W2_r[...], dn, preferred_element_type=jnp.float32)
        h = jnp.maximum(h + b2_r[...][None, :], 0.0)
        out = lax.dot_general(gmf, Wf_r[0:MF_DIM, :], dn,
                              preferred_element_type=jnp.float32)
        out = out + lax.dot_general(h, Wf_r[MF_DIM:2 * MF_DIM, :], dn,
                                    preferred_element_type=jnp.float32)
        out_r[...] = out + bf_r[...][None, :]

    return pl.pallas_call(
        body,
        out_shape=jax.ShapeDtypeStruct((B, 1), jnp.float32),
    )(umf, imf, umlp, imlp, W1, b1, W2, b2, Wf, bf)


def kernel(user, item, user_mf_emb, item_mf_emb, user_mlp_emb, item_mlp_emb,
           W1, b1, W2, b2, Wf, bf):
    user3 = user.astype(jnp.int32).reshape(NW, NCH, CH)
    item3 = item.astype(jnp.int32).reshape(NW, NCH, CH)
    umf, imf, umlp, imlp = _sc_gather(
        user3, item3, user_mf_emb, item_mf_emb, user_mlp_emb, item_mlp_emb)
    out = _tc_dense(umf, imf, umlp, imlp, W1, b1, W2, b2, Wf, bf)
    return jnp.squeeze(out, axis=-1)


# speedup vs baseline: 7.5543x; 7.5543x over previous
import functools

import jax
import jax.numpy as jnp
from jax import lax
from jax.experimental import pallas as pl
from jax.experimental.pallas import tpu as pltpu
from jax.experimental.pallas import tpu_sc as plsc

B = 16384
NUM_CORES = 2
NUM_SUBCORES = 16
NW = NUM_CORES * NUM_SUBCORES
BPW = B // NW
WAVE = 4
NPAIR = BPW // (2 * WAVE)
LANES = 128

MF_DIM = 8
MLP_IN = 16


def _sc_gather(user, item, umf_t, imf_t, umlp_t, imlp_t):
    mesh = plsc.VectorSubcoreMesh(core_axis_name="c", subcore_axis_name="s")

    @functools.partial(
        pl.kernel,
        mesh=mesh,
        compiler_params=pltpu.CompilerParams(use_tc_tiling_on_sc=True,
                                             needs_layout_passes=False),
        out_type=[
            jax.ShapeDtypeStruct((MF_DIM, B), jnp.float32),
            jax.ShapeDtypeStruct((MF_DIM, B), jnp.float32),
            jax.ShapeDtypeStruct((MLP_IN, B), jnp.float32),
            jax.ShapeDtypeStruct((MLP_IN, B), jnp.float32),
        ],
        scratch_types=[
            pltpu.VMEM((BPW + 16,), jnp.int32),
            pltpu.VMEM((BPW + 16,), jnp.int32),
            pltpu.VMEM((2, WAVE, MF_DIM, LANES), jnp.float32),
            pltpu.VMEM((2, WAVE, MF_DIM, LANES), jnp.float32),
            pltpu.VMEM((2, WAVE, MLP_IN, LANES), jnp.float32),
            pltpu.VMEM((2, WAVE, MLP_IN, LANES), jnp.float32),
            pltpu.VMEM((MF_DIM, BPW), jnp.float32),
            pltpu.VMEM((MF_DIM, BPW), jnp.float32),
            pltpu.VMEM((MLP_IN, BPW), jnp.float32),
            pltpu.VMEM((MLP_IN, BPW), jnp.float32),
            pltpu.SemaphoreType.DMA,
            pltpu.SemaphoreType.DMA,
        ],
    )
    def gather_kernel(user_h, item_h, umf_h, imf_h, umlp_h, imlp_h,
                      umf_o, imf_o, umlp_o, imlp_o,
                      uidx, iidx, umf_r, imf_r, umlp_r, imlp_r,
                      umf_v, imf_v, umlp_v, imlp_v,
                      semA, semB):
        wid = lax.axis_index("s") * NUM_CORES + lax.axis_index("c")
        base = wid * BPW
        pltpu.sync_copy(user_h.at[pl.ds(base, BPW)], uidx.at[pl.ds(0, BPW)])
        pltpu.sync_copy(item_h.at[pl.ds(base, BPW)], iidx.at[pl.ds(0, BPW)])

        iota = lax.iota(jnp.int32, 16)
        mf_rows = lax.rem(iota, MF_DIM)
        mf_mask = iota < MF_DIM
        sems = (semA, semB)

        def fire(buf, uv, iv, qoff):
            sem = sems[buf]
            for q in range(WAVE):
                u = uv[qoff + q]
                i = iv[qoff + q]
                ua = pl.multiple_of((u // LANES) * LANES, LANES)
                ia = pl.multiple_of((i // LANES) * LANES, LANES)
                pltpu.async_copy(umf_h.at[:, pl.ds(ua, LANES)],
                                 umf_r.at[buf, q], sem)
                pltpu.async_copy(imf_h.at[:, pl.ds(ia, LANES)],
                                 imf_r.at[buf, q], sem)
                pltpu.async_copy(umlp_h.at[:, pl.ds(ua, LANES)],
                                 umlp_r.at[buf, q], sem)
                pltpu.async_copy(imlp_h.at[:, pl.ds(ia, LANES)],
                                 imlp_r.at[buf, q], sem)

        def extract(buf, uv, iv, qoff, s0):
            for q in range(WAVE):
                ul = lax.rem(uv[qoff + q], LANES)
                il = lax.rem(iv[qoff + q], LANES)
                scol = jnp.full((16,), s0 + q, jnp.int32)
                vals = plsc.load_gather(
                    umf_r.at[buf, q],
                    [mf_rows, jnp.full((16,), ul, jnp.int32)], mask=mf_mask)
                plsc.store_scatter(umf_v, [mf_rows, scol], vals, mask=mf_mask)
                vals = plsc.load_gather(
                    imf_r.at[buf, q],
                    [mf_rows, jnp.full((16,), il, jnp.int32)], mask=mf_mask)
                plsc.store_scatter(imf_v, [mf_rows, scol], vals, mask=mf_mask)
                vals = plsc.load_gather(
                    umlp_r.at[buf, q], [iota, jnp.full((16,), ul, jnp.int32)])
                plsc.store_scatter(umlp_v, [iota, scol], vals)
                vals = plsc.load_gather(
                    imlp_r.at[buf, q], [iota, jnp.full((16,), il, jnp.int32)])
                plsc.store_scatter(imlp_v, [iota, scol], vals)

        def wait_extract(buf, uv, iv, qoff, s0):
            sem = sems[buf]
            for t_r, t_d in ((umf_r, MF_DIM), (imf_r, MF_DIM),
                             (umlp_r, MLP_IN), (imlp_r, MLP_IN)):
                for q in range(WAVE):
                    pltpu.make_async_copy(
                        umf_h.at[:, pl.ds(0, LANES)] if t_d == MF_DIM
                        else umlp_h.at[:, pl.ds(0, LANES)],
                        t_r.at[buf, q], sem).wait()
            extract(buf, uv, iv, qoff, s0)

        uv0 = uidx[pl.ds(0, 16)]
        iv0 = iidx[pl.ds(0, 16)]
        fire(0, uv0, iv0, 0)

        def body(p, carry):
            uv, iv = carry
            fire(1, uv, iv, WAVE)
            wait_extract(0, uv, iv, 0, p * 2 * WAVE)
            uv2 = uidx[pl.ds((p + 1) * 2 * WAVE, 16)]
            iv2 = iidx[pl.ds((p + 1) * 2 * WAVE, 16)]
            fire(0, uv2, iv2, 0)
            wait_extract(1, uv, iv, WAVE, p * 2 * WAVE + WAVE)
            return (uv2, iv2)

        uvL, ivL = lax.fori_loop(0, NPAIR - 1, body, (uv0, iv0))
        fire(1, uvL, ivL, WAVE)
        wait_extract(0, uvL, ivL, 0, (NPAIR - 1) * 2 * WAVE)
        wait_extract(1, uvL, ivL, WAVE, (NPAIR - 1) * 2 * WAVE + WAVE)

        out_sl = pl.ds(base, BPW)
        pltpu.sync_copy(umf_v, umf_o.at[:, out_sl])
        pltpu.sync_copy(imf_v, imf_o.at[:, out_sl])
        pltpu.sync_copy(umlp_v, umlp_o.at[:, out_sl])
        pltpu.sync_copy(imlp_v, imlp_o.at[:, out_sl])

    return gather_kernel(user, item, umf_t, imf_t, umlp_t, imlp_t)


def _tc_dense(umf, imf, umlp, imlp, W1, b1, W2, b2, Wf, bf):
    dn = (((0,), (0,)), ((), ()))

    def body(umf_r, imf_r, umlp_r, imlp_r, W1_r, b1_r, W2_r, b2_r, Wf_r, bf_r,
             out_r):
        gmf = umf_r[...] * imf_r[...]
        h = lax.dot_general(W1_r[0:MLP_IN, :], umlp_r[...], dn,
                            preferred_element_type=jnp.float32)
        h = h + lax.dot_general(W1_r[MLP_IN:2 * MLP_IN, :], imlp_r[...], dn,
                                preferred_element_type=jnp.float32)
        h = jnp.maximum(h + b1_r[...][:, None], 0.0)
        h = lax.dot_general(W2_r[...], h, dn, preferred_element_type=jnp.float32)
        h = jnp.maximum(h + b2_r[...][:, None], 0.0)
        o = lax.dot_general(Wf_r[0:MF_DIM, :], gmf, dn,
                            preferred_element_type=jnp.float32)
        o = o + lax.dot_general(Wf_r[MF_DIM:2 * MF_DIM, :], h, dn,
                                preferred_element_type=jnp.float32)
        out_r[...] = o + bf_r[...][:, None]

    return pl.pallas_call(
        body,
        out_shape=jax.ShapeDtypeStruct((1, B), jnp.float32),
    )(umf, imf, umlp, imlp, W1, b1, W2, b2, Wf, bf)


def kernel(user, item, user_mf_emb, item_mf_emb, user_mlp_emb, item_mlp_emb,
           W1, b1, W2, b2, Wf, bf):
    user = user.astype(jnp.int32)
    item = item.astype(jnp.int32)
    umf, imf, umlp, imlp = _sc_gather(
        user, item, user_mf_emb.T, item_mf_emb.T,
        user_mlp_emb.T, item_mlp_emb.T)
    out = _tc_dense(umf, imf, umlp, imlp, W1, b1, W2, b2, Wf, bf)
    return jnp.squeeze(out, axis=0)


# speedup vs baseline: 7.9719x; 1.0553x over previous
import functools

import jax
import jax.numpy as jnp
from jax import lax
from jax.experimental import pallas as pl
from jax.experimental.pallas import tpu as pltpu
from jax.experimental.pallas import tpu_sc as plsc

B = 16384
NUM_CORES = 2
NUM_SUBCORES = 16
NW = NUM_CORES * NUM_SUBCORES
BPW = B // NW
WAVE = 8
NPAIR = BPW // (2 * WAVE)
LANES = 128

MF_DIM = 8
MLP_IN = 16


def _sc_gather(user, item, umf_t, imf_t, umlp_t, imlp_t):
    mesh = plsc.VectorSubcoreMesh(core_axis_name="c", subcore_axis_name="s")

    @functools.partial(
        pl.kernel,
        mesh=mesh,
        compiler_params=pltpu.CompilerParams(use_tc_tiling_on_sc=True,
                                             needs_layout_passes=False),
        out_type=[
            jax.ShapeDtypeStruct((MF_DIM, B), jnp.float32),
            jax.ShapeDtypeStruct((MF_DIM, B), jnp.float32),
            jax.ShapeDtypeStruct((MLP_IN, B), jnp.float32),
            jax.ShapeDtypeStruct((MLP_IN, B), jnp.float32),
        ],
        scratch_types=[
            pltpu.VMEM((BPW + 16,), jnp.int32),
            pltpu.VMEM((BPW + 16,), jnp.int32),
            pltpu.VMEM((2, WAVE, MF_DIM, LANES), jnp.float32),
            pltpu.VMEM((2, WAVE, MF_DIM, LANES), jnp.float32),
            pltpu.VMEM((2, WAVE, MLP_IN, LANES), jnp.float32),
            pltpu.VMEM((2, WAVE, MLP_IN, LANES), jnp.float32),
            pltpu.VMEM((MF_DIM, BPW), jnp.float32),
            pltpu.VMEM((MF_DIM, BPW), jnp.float32),
            pltpu.VMEM((MLP_IN, BPW), jnp.float32),
            pltpu.VMEM((MLP_IN, BPW), jnp.float32),
            pltpu.SemaphoreType.DMA,
            pltpu.SemaphoreType.DMA,
        ],
    )
    def gather_kernel(user_h, item_h, umf_h, imf_h, umlp_h, imlp_h,
                      umf_o, imf_o, umlp_o, imlp_o,
                      uidx, iidx, umf_r, imf_r, umlp_r, imlp_r,
                      umf_v, imf_v, umlp_v, imlp_v,
                      semA, semB):
        wid = lax.axis_index("s") * NUM_CORES + lax.axis_index("c")
        base = wid * BPW
        pltpu.sync_copy(user_h.at[pl.ds(base, BPW)], uidx.at[pl.ds(0, BPW)])
        pltpu.sync_copy(item_h.at[pl.ds(base, BPW)], iidx.at[pl.ds(0, BPW)])

        iota = lax.iota(jnp.int32, 16)
        mf_rows = lax.rem(iota, MF_DIM)
        mf_mask = iota < MF_DIM
        sems = (semA, semB)

        def fire(buf, uv, iv, qoff):
            sem = sems[buf]
            for q in range(WAVE):
                u = uv[qoff + q]
                i = iv[qoff + q]
                ua = pl.multiple_of((u // LANES) * LANES, LANES)
                ia = pl.multiple_of((i // LANES) * LANES, LANES)
                pltpu.async_copy(umf_h.at[:, pl.ds(ua, LANES)],
                                 umf_r.at[buf, q], sem)
                pltpu.async_copy(imf_h.at[:, pl.ds(ia, LANES)],
                                 imf_r.at[buf, q], sem)
                pltpu.async_copy(umlp_h.at[:, pl.ds(ua, LANES)],
                                 umlp_r.at[buf, q], sem)
                pltpu.async_copy(imlp_h.at[:, pl.ds(ia, LANES)],
                                 imlp_r.at[buf, q], sem)

        def extract(buf, uv, iv, qoff, s0):
            for q in range(WAVE):
                ul = lax.rem(uv[qoff + q], LANES)
                il = lax.rem(iv[qoff + q], LANES)
                scol = jnp.full((16,), s0 + q, jnp.int32)
                vals = plsc.load_gather(
                    umf_r.at[buf, q],
                    [mf_rows, jnp.full((16,), ul, jnp.int32)], mask=mf_mask)
                plsc.store_scatter(umf_v, [mf_rows, scol], vals, mask=mf_mask)
                vals = plsc.load_gather(
                    imf_r.at[buf, q],
                    [mf_rows, jnp.full((16,), il, jnp.int32)], mask=mf_mask)
                plsc.store_scatter(imf_v, [mf_rows, scol], vals, mask=mf_mask)
                vals = plsc.load_gather(
                    umlp_r.at[buf, q], [iota, jnp.full((16,), ul, jnp.int32)])
                plsc.store_scatter(umlp_v, [iota, scol], vals)
                vals = plsc.load_gather(
                    imlp_r.at[buf, q], [iota, jnp.full((16,), il, jnp.int32)])
                plsc.store_scatter(imlp_v, [iota, scol], vals)

        def wait_extract(buf, uv, iv, qoff, s0):
            sem = sems[buf]
            for t_r, t_d in ((umf_r, MF_DIM), (imf_r, MF_DIM),
                             (umlp_r, MLP_IN), (imlp_r, MLP_IN)):
                for q in range(WAVE):
                    pltpu.make_async_copy(
                        umf_h.at[:, pl.ds(0, LANES)] if t_d == MF_DIM
                        else umlp_h.at[:, pl.ds(0, LANES)],
                        t_r.at[buf, q], sem).wait()
            extract(buf, uv, iv, qoff, s0)

        uv0 = uidx[pl.ds(0, 16)]
        iv0 = iidx[pl.ds(0, 16)]
        fire(0, uv0, iv0, 0)

        def body(p, carry):
            uv, iv = carry
            fire(1, uv, iv, WAVE)
            wait_extract(0, uv, iv, 0, p * 2 * WAVE)
            uv2 = uidx[pl.ds((p + 1) * 2 * WAVE, 16)]
            iv2 = iidx[pl.ds((p + 1) * 2 * WAVE, 16)]
            fire(0, uv2, iv2, 0)
            wait_extract(1, uv, iv, WAVE, p * 2 * WAVE + WAVE)
            return (uv2, iv2)

        uvL, ivL = lax.fori_loop(0, NPAIR - 1, body, (uv0, iv0))
        fire(1, uvL, ivL, WAVE)
        wait_extract(0, uvL, ivL, 0, (NPAIR - 1) * 2 * WAVE)
        wait_extract(1, uvL, ivL, WAVE, (NPAIR - 1) * 2 * WAVE + WAVE)

        out_sl = pl.ds(base, BPW)
        pltpu.sync_copy(umf_v, umf_o.at[:, out_sl])
        pltpu.sync_copy(imf_v, imf_o.at[:, out_sl])
        pltpu.sync_copy(umlp_v, umlp_o.at[:, out_sl])
        pltpu.sync_copy(imlp_v, imlp_o.at[:, out_sl])

    return gather_kernel(user, item, umf_t, imf_t, umlp_t, imlp_t)


def _tc_dense(umf, imf, umlp, imlp, W1, b1, W2, b2, Wf, bf):
    dn = (((0,), (0,)), ((), ()))

    def body(umf_r, imf_r, umlp_r, imlp_r, W1_r, b1_r, W2_r, b2_r, Wf_r, bf_r,
             out_r):
        gmf = umf_r[...] * imf_r[...]
        h = lax.dot_general(W1_r[0:MLP_IN, :], umlp_r[...], dn,
                            preferred_element_type=jnp.float32)
        h = h + lax.dot_general(W1_r[MLP_IN:2 * MLP_IN, :], imlp_r[...], dn,
                                preferred_element_type=jnp.float32)
        h = jnp.maximum(h + b1_r[...][:, None], 0.0)
        h = lax.dot_general(W2_r[...], h, dn, preferred_element_type=jnp.float32)
        h = jnp.maximum(h + b2_r[...][:, None], 0.0)
        o = lax.dot_general(Wf_r[0:MF_DIM, :], gmf, dn,
                            preferred_element_type=jnp.float32)
        o = o + lax.dot_general(Wf_r[MF_DIM:2 * MF_DIM, :], h, dn,
                                preferred_element_type=jnp.float32)
        out_r[...] = o + bf_r[...][:, None]

    return pl.pallas_call(
        body,
        out_shape=jax.ShapeDtypeStruct((1, B), jnp.float32),
    )(umf, imf, umlp, imlp, W1, b1, W2, b2, Wf, bf)


def kernel(user, item, user_mf_emb, item_mf_emb, user_mlp_emb, item_mlp_emb,
           W1, b1, W2, b2, Wf, bf):
    user = user.astype(jnp.int32)
    item = item.astype(jnp.int32)
    umf, imf, umlp, imlp = _sc_gather(
        user, item, user_mf_emb.T, item_mf_emb.T,
        user_mlp_emb.T, item_mlp_emb.T)
    out = _tc_dense(umf, imf, umlp, imlp, W1, b1, W2, b2, Wf, bf)
    return jnp.squeeze(out, axis=0)
